# baseline (speedup 1.0000x reference)
import functools

import jax
import jax.numpy as jnp
from jax import lax
from jax.experimental import pallas as pl
from jax.experimental.pallas import tpu as pltpu
from jax.experimental.pallas import tpu_sc as plsc

_VOCAB = 1000000
_EMBED = 64
_BATCH = 4096
_SEQ = 200
_NC = 2
_NS = 16
_NW = _NC * _NS
_ROWS_W = _BATCH // _NW
_NB = 4
_L = 2
_GROUPS = _ROWS_W // _NB

_mesh = plsc.VectorSubcoreMesh(core_axis_name="c", subcore_axis_name="s")


@functools.partial(
    pl.kernel,
    mesh=_mesh,
    compiler_params=pltpu.CompilerParams(use_tc_tiling_on_sc=False),
    out_type=jax.ShapeDtypeStruct((_BATCH * _SEQ, 2 * _EMBED), jnp.float32),
    scratch_types=[
        pltpu.VMEM((_ROWS_W, _SEQ), jnp.int32),
        pltpu.VMEM((_NB, _SEQ, _EMBED), jnp.float32),
        pltpu.SemaphoreType.DMA((_NB,)),
        pltpu.SemaphoreType.DMA((_NB,)),
    ],
)
def _embed_lookup(x_hbm, table_hbm, out_hbm, idx_v, rows_v, gat_sem, out_sem):
    wid = lax.axis_index("s") * _NC + lax.axis_index("c")
    wrow = pl.multiple_of(wid * _ROWS_W, 8)
    pltpu.sync_copy(x_hbm.at[pl.ds(wrow, _ROWS_W)], idx_v)

    def start_gather(b, r):
        pltpu.make_async_copy(
            table_hbm.at[idx_v.at[r]], rows_v.at[b], gat_sem.at[b]
        ).start()

    def wait_gather(b):
        pltpu.make_async_copy(
            table_hbm.at[idx_v.at[0]], rows_v.at[b], gat_sem.at[b]
        ).wait()

    def start_out(b, r):
        pltpu.make_async_copy(
            rows_v.at[b],
            out_hbm.at[pl.ds((wrow + r) * _SEQ, _SEQ), pl.ds(0, _EMBED)],
            out_sem.at[b],
        ).start()

    def wait_out(b):
        pltpu.make_async_copy(
            rows_v.at[b],
            out_hbm.at[pl.ds(wrow * _SEQ, _SEQ), pl.ds(0, _EMBED)],
            out_sem.at[b],
        ).wait()

    def body(g, carry):
        for b in range(_NB):
            r = g * _NB + b
            @pl.when(g >= 1)
            def _():
                wait_out(b)

            start_gather(b, r)

            b2 = (b - _L) % _NB
            r2 = r - _L

            @pl.when(r2 >= 0)
            def _():
                wait_gather(b2)
                start_out(b2, r2)

        return carry

    lax.fori_loop(0, _GROUPS, body, 0)

    for k in range(_L):
        r2 = _ROWS_W - _L + k
        b2 = r2 % _NB
        wait_gather(b2)
        start_out(b2, r2)
    for b in range(_NB):
        wait_out(b)


def kernel(x, table):
    out = _embed_lookup(x, table)
    return out[:, :_EMBED].reshape(_BATCH, _SEQ, _EMBED)
